# baseline (speedup 1.0000x reference)
import functools

import jax
import jax.numpy as jnp
from jax import lax
from jax.experimental import pallas as pl
from jax.experimental.pallas import tpu as pltpu
from jax.experimental.pallas import tpu_sc as plsc

_EMBED = 64
_NFEAT = 13
_W = _NFEAT * _EMBED
_B = 16384
_NCORES = 2
_NSUB = 16
_NW = _NCORES * _NSUB
_ROWS_PER_W = _B // _NW
_R = 16
_NCHUNK = _ROWS_PER_W // _R
_NBUF = 2


def _copy_plan():
    plan = []
    for j in range(_NFEAT):
        if j % 2 == 0:
            plan.append((0, j, 0, j // 2))
            plan.append((1, j, 1, 6 + j // 2))
        else:
            plan.append((0, j, 1, (j - 1) // 2))
            plan.append((1, j, 0, 7 + (j - 1) // 2))
    return plan


_PLAN = _copy_plan()


def _shuffle_chunk(in0, in1, ev, od):
    srcs = (in0, in1)
    dsts = (ev, od)

    def row(r, carry):
        for si, sb, di, db in _PLAN:
            for v in range(_EMBED // 16):
                s = sb * _EMBED + v * 16
                d = db * _EMBED + v * 16
                dsts[di][r, pl.ds(d, 16)] = srcs[si][r, pl.ds(s, 16)]
        return carry

    lax.fori_loop(0, _R, row, 0)


def _body(v0, v1, ev_hbm, od_hbm,
          in0_a, in0_b, in1_a, in1_b, ev_a, ev_b, od_a, od_b,
          sin_a, sin_b, sout_a, sout_b):
    wid = lax.axis_index("s") * _NCORES + lax.axis_index("c")
    base = wid * _ROWS_PER_W

    in0 = (in0_a, in0_b)
    in1 = (in1_a, in1_b)
    ev = (ev_a, ev_b)
    od = (od_a, od_b)
    sin = (sin_a, sin_b)
    sout = (sout_a, sout_b)

    def rows(k):
        return pl.ds(base + k * _R, _R)

    for b in range(_NBUF):
        pltpu.async_copy(v0.at[rows(b)], in0[b], sin[b])
        pltpu.async_copy(v1.at[rows(b)], in1[b], sin[b])

    def step(g, carry):
        for b in range(_NBUF):
            k = g * _NBUF + b
            pltpu.make_async_copy(v0.at[rows(k)], in0[b], sin[b]).wait()
            pltpu.make_async_copy(v1.at[rows(k)], in1[b], sin[b]).wait()

            @pl.when(k >= _NBUF)
            def _drain_out():
                pltpu.make_async_copy(ev[b], ev_hbm.at[rows(k)], sout[b]).wait()
                pltpu.make_async_copy(od[b], od_hbm.at[rows(k)], sout[b]).wait()

            _shuffle_chunk(in0[b], in1[b], ev[b], od[b])

            pltpu.async_copy(ev[b], ev_hbm.at[rows(k)], sout[b])
            pltpu.async_copy(od[b], od_hbm.at[rows(k)], sout[b])

            @pl.when(k + _NBUF < _NCHUNK)
            def _refill():
                pltpu.async_copy(v0.at[rows(k + _NBUF)], in0[b], sin[b])
                pltpu.async_copy(v1.at[rows(k + _NBUF)], in1[b], sin[b])
        return carry

    lax.fori_loop(0, _NCHUNK // _NBUF, step, 0)

    for b in range(_NBUF):
        pltpu.make_async_copy(ev[b], ev_hbm.at[rows(0)], sout[b]).wait()
        pltpu.make_async_copy(od[b], od_hbm.at[rows(0)], sout[b]).wait()


@functools.partial(
    pl.kernel,
    out_type=(
        jax.ShapeDtypeStruct((_B, _W), jnp.float32),
        jax.ShapeDtypeStruct((_B, _W), jnp.float32),
    ),
    mesh=plsc.VectorSubcoreMesh(core_axis_name="c", subcore_axis_name="s"),
    scratch_types=[
        pltpu.VMEM((_R, _W), jnp.float32),
        pltpu.VMEM((_R, _W), jnp.float32),
        pltpu.VMEM((_R, _W), jnp.float32),
        pltpu.VMEM((_R, _W), jnp.float32),
        pltpu.VMEM((_R, _W), jnp.float32),
        pltpu.VMEM((_R, _W), jnp.float32),
        pltpu.VMEM((_R, _W), jnp.float32),
        pltpu.VMEM((_R, _W), jnp.float32),
        pltpu.SemaphoreType.DMA,
        pltpu.SemaphoreType.DMA,
        pltpu.SemaphoreType.DMA,
        pltpu.SemaphoreType.DMA,
    ],
)
def _regroup(v0, v1, ev_hbm, od_hbm, *scratch):
    _body(v0, v1, ev_hbm, od_hbm, *scratch)


def kernel(values0, values1):
    return _regroup(values0, values1)
